# baseline (speedup 1.0000x reference)
import functools

import jax
import jax.numpy as jnp
from jax import lax
from jax.experimental import pallas as pl
from jax.experimental.pallas import tpu as pltpu
from jax.experimental.pallas import tpu_sc as plsc

AUG_THRESHOLD = 1024
CPOS = 128
NB = 4
CW = CPOS * 128
NCAND = 13


def _sc_body(B, L, D, x_hbm, lens_hbm, o_hbm, b0, b1, b2, b3, lens_v,
             sem_in, sem_out):
    bufs = (b0, b1, b2, b3)
    half = L // 2
    nchunk = half // CPOS
    c = lax.axis_index("c")
    s = lax.axis_index("s")
    row = s
    base0 = c * half

    pltpu.sync_copy(lens_hbm, lens_v)
    row_idx = jnp.full((16,), row, jnp.int32)
    ln_vec = plsc.load_gather(lens_v, [row_idx])
    lim_vec = jnp.where(ln_vec > AUG_THRESHOLD, ln_vec, 0)

    row_off = row * L * D

    def in_copy(g, j):
        off = row_off + (base0 + g * CPOS) * D
        return pltpu.make_async_copy(
            x_hbm.at[pl.ds(off, CW)], bufs[j], sem_in.at[j]
        )

    def out_copy(g, j):
        off = row_off + (base0 + g * CPOS) * D
        return pltpu.make_async_copy(
            bufs[j], o_hbm.at[pl.ds(off, CW)], sem_out.at[j]
        )

    zeros16 = jnp.zeros((16,), jnp.float32)

    for g in range(min(NB, nchunk)):
        in_copy(g, g % NB).start()

    for g in range(nchunk):
        j = g % NB
        if g >= 2 and g + 2 < nchunk:
            out_copy(g - 2, (g - 2) % NB).wait()
            in_copy(g + 2, (g + 2) % NB).start()
        in_copy(g, j).wait()

        base = base0 + g * CPOS
        first = lax.rem(10 - lax.rem(base, 10), 10)
        buf = bufs[j]

        def zero_body(i, _):
            offc = first + 10 * i
            p_vec = jnp.full((16,), base + offc, jnp.int32)
            off_vec = jnp.full((16,), offc, jnp.int32)
            cond = (off_vec < CPOS) & (p_vec < lim_vec)
            addr = jnp.minimum(offc, CPOS - 1) * D
            for k in range(8):
                sl = pl.ds(addr + 16 * k, 16)
                buf[sl] = jnp.where(cond, zeros16, buf[sl])
            return 0

        lax.fori_loop(0, NCAND, zero_body, 0)

        out_copy(g, j).start()

    for g in range(max(nchunk - 4, 0), nchunk):
        out_copy(g, g % NB).wait()


def kernel(sequences, seq_lens):
    B, L, D = sequences.shape
    x1 = sequences.reshape(-1)
    mesh = plsc.VectorSubcoreMesh(core_axis_name="c", subcore_axis_name="s")
    kern = functools.partial(
        pl.kernel,
        mesh=mesh,
        out_type=jax.ShapeDtypeStruct((B * L * D,), jnp.float32),
        scratch_types=[
            pltpu.VMEM((CW,), jnp.float32),
            pltpu.VMEM((CW,), jnp.float32),
            pltpu.VMEM((CW,), jnp.float32),
            pltpu.VMEM((CW,), jnp.float32),
            pltpu.VMEM((16,), jnp.int32),
            pltpu.SemaphoreType.DMA((NB,)),
            pltpu.SemaphoreType.DMA((NB,)),
        ],
        compiler_params=pltpu.CompilerParams(needs_layout_passes=False),
    )(functools.partial(_sc_body, B, L, D))
    out = kern(x1, seq_lens)
    return out.reshape(B, L, D), seq_lens


# speedup vs baseline: 1.0249x; 1.0249x over previous
import functools

import jax
import jax.numpy as jnp
from jax import lax
from jax.experimental import pallas as pl
from jax.experimental.pallas import tpu as pltpu
from jax.experimental.pallas import tpu_sc as plsc

AUG_THRESHOLD = 1024
CPOS = 256
NB = 3
CW = CPOS * 128
NCAND = 26


def _sc_body(B, L, D, x_hbm, lens_hbm, o_hbm, b0, b1, b2, lens_v,
             sem_in, sem_out):
    bufs = (b0, b1, b2)
    half = L // 2
    nchunk = half // CPOS
    c = lax.axis_index("c")
    s = lax.axis_index("s")
    row = s
    base0 = c * half

    pltpu.sync_copy(lens_hbm, lens_v)
    row_idx = jnp.full((16,), row, jnp.int32)
    ln_vec = plsc.load_gather(lens_v, [row_idx])
    lim_vec = jnp.where(ln_vec > AUG_THRESHOLD, ln_vec, 0)

    row_off = row * L * D

    def in_copy(g, j):
        off = row_off + (base0 + g * CPOS) * D
        return pltpu.make_async_copy(
            x_hbm.at[pl.ds(off, CW)], bufs[j], sem_in.at[j]
        )

    def out_copy(g, j):
        off = row_off + (base0 + g * CPOS) * D
        return pltpu.make_async_copy(
            bufs[j], o_hbm.at[pl.ds(off, CW)], sem_out.at[j]
        )

    zeros16 = jnp.zeros((16,), jnp.float32)

    for g in range(min(NB, nchunk)):
        in_copy(g, g % NB).start()

    for g in range(nchunk):
        j = g % NB
        if g >= 1 and g + 2 < nchunk:
            out_copy(g - 1, (g - 1) % NB).wait()
            in_copy(g + 2, (g + 2) % NB).start()
        in_copy(g, j).wait()

        base = base0 + g * CPOS
        first = lax.rem(10 - lax.rem(base, 10), 10)
        buf = bufs[j]

        def zero_body(i, _):
            offc = first + 10 * i
            p_vec = jnp.full((16,), base + offc, jnp.int32)
            off_vec = jnp.full((16,), offc, jnp.int32)
            cond = (off_vec < CPOS) & (p_vec < lim_vec)
            addr = jnp.minimum(offc, CPOS - 1) * D
            for k in range(8):
                sl = pl.ds(addr + 16 * k, 16)
                buf[sl] = jnp.where(cond, zeros16, buf[sl])
            return 0

        lax.fori_loop(0, NCAND, zero_body, 0)

        out_copy(g, j).start()

    for g in range(max(nchunk - 3, 0), nchunk):
        out_copy(g, g % NB).wait()


def kernel(sequences, seq_lens):
    B, L, D = sequences.shape
    x1 = sequences.reshape(-1)
    mesh = plsc.VectorSubcoreMesh(core_axis_name="c", subcore_axis_name="s")
    kern = functools.partial(
        pl.kernel,
        mesh=mesh,
        out_type=jax.ShapeDtypeStruct((B * L * D,), jnp.float32),
        scratch_types=[
            pltpu.VMEM((CW,), jnp.float32),
            pltpu.VMEM((CW,), jnp.float32),
            pltpu.VMEM((CW,), jnp.float32),
            pltpu.VMEM((16,), jnp.int32),
            pltpu.SemaphoreType.DMA((NB,)),
            pltpu.SemaphoreType.DMA((NB,)),
        ],
        compiler_params=pltpu.CompilerParams(needs_layout_passes=False),
    )(functools.partial(_sc_body, B, L, D))
    out = kern(x1, seq_lens)
    return out.reshape(B, L, D), seq_lens
